# baseline (speedup 1.0000x reference)
import functools

import jax
import jax.numpy as jnp
from jax import lax
from jax.experimental import pallas as pl
from jax.experimental.pallas import tpu as pltpu
from jax.experimental.pallas import tpu_sc as plsc

N = 10000
E = 320000
NBT = 4
F = 128
EPS = 1e-5


def _tc1_body(a0_ref, a1_ref, g_ref, b_ref, w_ref, h_ref):
    x = jnp.concatenate([a0_ref[...], a1_ref[...]], axis=-1)
    m = jnp.mean(x, axis=0, keepdims=True)
    xc = x - m
    v = jnp.mean(xc * xc, axis=0, keepdims=True)
    xn = xc * lax.rsqrt(v + EPS) * g_ref[...][None, :] + b_ref[...][None, :]
    e = jnp.where(xn > 0, xn, (jnp.exp(xn) - 1.0))
    h_ref[...] = jnp.dot(e, w_ref[...], preferred_element_type=jnp.float32)


def _tc1(a0, a1, g, b, w):
    return pl.pallas_call(
        _tc1_body,
        out_shape=jax.ShapeDtypeStruct((N, F), jnp.float32),
    )(a0, a1, g, b, w)


NC, NS = 2, 16
CN = 2500
NCHUNK = N // CN
PASSES = NCHUNK // NC
ROWS = CN * NBT
ACC_ROWS = 10240
JUNK = ROWS
EPT = E // NS
SB = 800
NSB = EPT // SB
GB = 128
RING = 16
NBUF = 3

_mesh = plsc.VectorSubcoreMesh(core_axis_name="c", subcore_axis_name="s")


@functools.partial(
    pl.kernel,
    out_type=jax.ShapeDtypeStruct((NCHUNK * ROWS, F), jnp.bfloat16),
    mesh=_mesh,
    scratch_types=[
        pltpu.VMEM_SHARED((ACC_ROWS, F), jnp.bfloat16),
        pltpu.VMEM((SB * 3,), jnp.int32),
        pltpu.VMEM((SB * 3,), jnp.int32),
        pltpu.VMEM((RING, GB), jnp.int32),
        pltpu.VMEM((RING, GB), jnp.int32),
        [pltpu.VMEM((GB, F), jnp.bfloat16)] * 3,
        [pltpu.SemaphoreType.DMA] * 3,
        [pltpu.SemaphoreType.DMA] * 3,
        pltpu.SemaphoreType.DMA,
        pltpu.SemaphoreType.DMA,
    ],
    compiler_params=pltpu.CompilerParams(needs_layout_passes=False,
                                         use_tc_tiling_on_sc=False),
)
def _sc_msgpass(h_hbm, bond_hbm, zeros_hbm, buf_hbm,
                acc, ev0, ev1, gidx, dslot, rows, gsem, asem,
                sem_e0, sem_e1):
    c = lax.axis_index("c")
    s = lax.axis_index("s")
    zrows = ACC_ROWS // NS
    orows8 = (ROWS // NS) // 8 * 8
    ev = (ev0, ev1)
    sem_e = (sem_e0, sem_e1)

    def _fire_stage(sb, which):
        off = (s * EPT + sb * SB) * 3
        pltpu.async_copy(bond_hbm.at[pl.ds(off, SB * 3)], ev[which],
                         sem_e[which])

    def _wait_stage(sb, which):
        off = (s * EPT + sb * SB) * 3
        pltpu.make_async_copy(bond_hbm.at[pl.ds(off, SB * 3)], ev[which],
                              sem_e[which]).wait()

    def _fire_gather(j):
        for b in range(NBUF):
            @pl.when(j % NBUF == b)
            def _(b=b):
                pltpu.async_copy(h_hbm.at[gidx.at[j % RING]], rows[b],
                                 gsem[b])

    def _wait_gather_fire_add(j):
        for b in range(NBUF):
            @pl.when(j % NBUF == b)
            def _(b=b):
                pltpu.make_async_copy(h_hbm.at[gidx.at[j % RING]], rows[b],
                                      gsem[b]).wait()
                pltpu.async_copy(rows[b], acc.at[dslot.at[j % RING]],
                                 asem[b], add=True)

    def _wait_add(j):
        for b in range(NBUF):
            @pl.when(j % NBUF == b)
            def _(b=b):
                pltpu.make_async_copy(rows[b], acc.at[dslot.at[j % RING]],
                                      asem[b]).wait()

    def _pass_body(p, _carry):
        chunk = p * NC + c
        lo = chunk * CN

        with jax.named_scope("sc_zero"):
            pltpu.sync_copy(zeros_hbm, acc.at[pl.ds(s * zrows, zrows)])
            plsc.subcore_barrier()

        ptrv = jnp.zeros((16,), jnp.int32)
        ptr = jnp.int32(0)
        nfired = jnp.int32(0)
        _fire_stage(0, 0)
        for sb in range(NSB):
            which = sb % 2
            if sb + 1 < NSB:
                _fire_stage(sb + 1, 1 - which)
            _wait_stage(sb, which)
            ev3 = ev[which]

            def _vec(i, ptrv, ev3=ev3):
                ridx = (i * 16 + lax.iota(jnp.int32, 16)) * 3
                b16 = plsc.load_gather(ev3, [ridx])
                e16 = plsc.load_gather(ev3, [ridx + 1])
                t16 = plsc.load_gather(ev3, [ridx + 2])
                m = (e16 >= lo) & (e16 < lo + CN)
                mi = m.astype(jnp.int32)
                pos = jnp.maximum(ptrv + jnp.cumsum(mi) - 1, 0)
                row = (pos // GB) % RING
                col = pos % GB
                d16 = (e16 - lo) * NBT + (t16 % NBT)
                plsc.store_scatter(gidx, [row, col], b16, mask=m)
                plsc.store_scatter(dslot, [row, col], d16, mask=m)
                return ptrv + plsc.all_reduce_population_count(m)

            with jax.named_scope("sc_filter"):
                ptrv = lax.fori_loop(0, SB // 16, _vec, ptrv)
            ptr = jnp.max(ptrv)

            def _adv(j, _):
                @pl.when(j >= 1)
                def _():
                    _wait_gather_fire_add(j - 1)

                @pl.when(j >= NBUF)
                def _():
                    _wait_add(j - NBUF)
                _fire_gather(j)
                return 0

            with jax.named_scope("sc_advance"):
                lax.fori_loop(nfired, ptr // GB, _adv, 0)
            nfired = ptr // GB

        nb = (ptr + GB - 1) // GB
        total = nb * GB

        def _pad(v, _):
            posv = ptr + v * 16 + lax.iota(jnp.int32, 16)
            m = posv < total
            row = (posv // GB) % RING
            col = posv % GB
            plsc.store_scatter(gidx, [row, col], jnp.zeros((16,), jnp.int32),
                               mask=m)
            jrow = JUNK + lax.rem(posv + s * 8, jnp.int32(ACC_ROWS - ROWS))
            plsc.store_scatter(dslot, [row, col], jrow, mask=m)
            return 0

        with jax.named_scope("sc_pad"):
            lax.fori_loop(0, GB // 16, _pad, 0)

        def _adv2(j, _):
            @pl.when(j >= 1)
            def _():
                _wait_gather_fire_add(j - 1)

            @pl.when(j >= NBUF)
            def _():
                _wait_add(j - NBUF)
            _fire_gather(j)
            return 0

        with jax.named_scope("sc_tail"):
            lax.fori_loop(nfired, nb, _adv2, 0)

            @pl.when(nb > 0)
            def _():
                _wait_gather_fire_add(nb - 1)

            def _drain(j, _):
                _wait_add(j)
                return 0

            lax.fori_loop(jnp.maximum(nb - NBUF, 0), nb, _drain, 0)
        plsc.subcore_barrier()

        with jax.named_scope("sc_flush"):
          pltpu.sync_copy(
            acc.at[pl.ds(s * orows8, orows8)],
            buf_hbm.at[pl.ds(chunk * ROWS + s * orows8, orows8)])

        @pl.when(s == NS - 1)
        def _():
            rem = ROWS - NS * orows8
            pltpu.sync_copy(
                acc.at[pl.ds(NS * orows8, rem)],
                buf_hbm.at[pl.ds(chunk * ROWS + NS * orows8, rem)])

        plsc.subcore_barrier()
        return 0

    lax.fori_loop(0, PASSES, _pass_body, 0)


BK = 1000
NBK = N // BK


def _stats_body(h_ref, b_ref, o_ref):
    i = pl.program_id(0)

    @pl.when(i == 0)
    def _():
        o_ref[...] = jnp.zeros_like(o_ref)

    x = jnp.concatenate([h_ref[...], b_ref[...].astype(jnp.float32)],
                        axis=-1)
    o_ref[0:1, :] += jnp.sum(x, axis=0, keepdims=True)
    o_ref[1:2, :] += jnp.sum(x * x, axis=0, keepdims=True)


def _tc2_stats(h, buf4):
    cd = F + NBT * F
    return pl.pallas_call(
        _stats_body,
        grid=(NBK,),
        in_specs=[
            pl.BlockSpec((BK, F), lambda i: (i, 0)),
            pl.BlockSpec((BK, NBT * F), lambda i: (i, 0)),
        ],
        out_specs=pl.BlockSpec((8, cd), lambda i: (0, 0)),
        out_shape=jax.ShapeDtypeStruct((8, cd), jnp.float32),
    )(h, buf4)


def _apply_body(h_ref, b_ref, st_ref, g_ref, bt_ref, w_ref, o_ref):
    x = jnp.concatenate([h_ref[...], b_ref[...].astype(jnp.float32)],
                        axis=-1)
    m = st_ref[0:1, :] * (1.0 / N)
    v = st_ref[1:2, :] * (1.0 / N) - m * m
    xn = (x - m) * lax.rsqrt(v + EPS) * g_ref[...][None, :] + bt_ref[...][None, :]
    e = jnp.where(xn > 0, xn, (jnp.exp(xn) - 1.0))
    o_ref[...] = jnp.dot(e, w_ref[...], preferred_element_type=jnp.float32)


def _tc2_apply(h, buf4, stats, g2, b2, w2):
    cd = F + NBT * F
    return pl.pallas_call(
        _apply_body,
        grid=(NBK,),
        in_specs=[
            pl.BlockSpec((BK, F), lambda i: (i, 0)),
            pl.BlockSpec((BK, NBT * F), lambda i: (i, 0)),
            pl.BlockSpec((8, cd), lambda i: (0, 0)),
            pl.BlockSpec((cd,), lambda i: (0,)),
            pl.BlockSpec((cd,), lambda i: (0,)),
            pl.BlockSpec((cd, F), lambda i: (0, 0)),
        ],
        out_specs=pl.BlockSpec((BK, F), lambda i: (i, 0)),
        out_shape=jax.ShapeDtypeStruct((N, F), jnp.float32),
    )(h, buf4, stats, g2, b2, w2)


def kernel(atom_features_list, bond_info, bn_gamma1, bn_beta1, W1,
           bn_gamma2, bn_beta2, W2):
    a0 = atom_features_list[0]
    a1 = atom_features_list[1]
    h = _tc1(a0, a1, bn_gamma1, bn_beta1, W1)

    zeros128 = jnp.zeros((ACC_ROWS // NS, F), jnp.bfloat16)
    buf = _sc_msgpass(h.astype(jnp.bfloat16), bond_info.reshape(E * 3),
                      zeros128)
    buf4 = buf.reshape(N, NBT * F)

    stats = _tc2_stats(h, buf4)
    return _tc2_apply(h, buf4, stats, bn_gamma2, bn_beta2, W2)


# speedup vs baseline: 1.5410x; 1.5410x over previous
import functools

import jax
import jax.numpy as jnp
from jax import lax
from jax.experimental import pallas as pl
from jax.experimental.pallas import tpu as pltpu
from jax.experimental.pallas import tpu_sc as plsc

N = 10000
E = 320000
NBT = 4
F = 128
EPS = 1e-5


def _tc1_body(a0_ref, a1_ref, g_ref, b_ref, w_ref, h_ref):
    x = jnp.concatenate([a0_ref[...], a1_ref[...]], axis=-1)
    m = jnp.mean(x, axis=0, keepdims=True)
    xc = x - m
    v = jnp.mean(xc * xc, axis=0, keepdims=True)
    xn = xc * lax.rsqrt(v + EPS) * g_ref[...][None, :] + b_ref[...][None, :]
    e = jnp.where(xn > 0, xn, (jnp.exp(xn) - 1.0))
    h_ref[...] = jnp.dot(e, w_ref[...], preferred_element_type=jnp.float32)


def _tc1(a0, a1, g, b, w):
    return pl.pallas_call(
        _tc1_body,
        out_shape=jax.ShapeDtypeStruct((N, F), jnp.float32),
    )(a0, a1, g, b, w)


NC, NS = 2, 16
CN = 2500
NCHUNK = N // CN
PASSES = NCHUNK // NC
ROWS = CN * NBT
ACC_ROWS = 10240
JUNK = ROWS
EPT = E // NS
SB = 800
NSB = EPT // SB
GB = 128
RING = 16
NBUF = 3

_mesh = plsc.VectorSubcoreMesh(core_axis_name="c", subcore_axis_name="s")


@functools.partial(
    pl.kernel,
    out_type=jax.ShapeDtypeStruct((NCHUNK * ROWS, F), jnp.bfloat16),
    mesh=_mesh,
    scratch_types=[
        pltpu.VMEM_SHARED((ACC_ROWS, F), jnp.bfloat16),
        [pltpu.VMEM((SB,), jnp.int32)] * 3,
        [pltpu.VMEM((SB,), jnp.int32)] * 3,
        pltpu.VMEM((RING, GB), jnp.int32),
        pltpu.VMEM((RING, GB), jnp.int32),
        [pltpu.VMEM((GB, F), jnp.bfloat16)] * 3,
        [pltpu.SemaphoreType.DMA] * 3,
        [pltpu.SemaphoreType.DMA] * 3,
        pltpu.SemaphoreType.DMA,
        pltpu.SemaphoreType.DMA,
    ],
    compiler_params=pltpu.CompilerParams(needs_layout_passes=False,
                                         use_tc_tiling_on_sc=False),
)
def _sc_msgpass(h_hbm, begin_hbm, end_hbm, bt_hbm, zeros_hbm, buf_hbm,
                acc, ev0, ev1, gidx, dslot, rows, gsem, asem,
                sem_e0, sem_e1):
    c = lax.axis_index("c")
    s = lax.axis_index("s")
    zrows = ACC_ROWS // NS
    orows8 = (ROWS // NS) // 8 * 8
    ev = (ev0, ev1)
    sem_e = (sem_e0, sem_e1)
    hbm3 = (begin_hbm, end_hbm, bt_hbm)

    def _fire_stage(sb, which):
        off = s * EPT + sb * SB
        for k in range(3):
            pltpu.async_copy(hbm3[k].at[pl.ds(off, SB)], ev[which][k],
                             sem_e[which])

    def _wait_stage(sb, which):
        off = s * EPT + sb * SB
        for k in range(3):
            pltpu.make_async_copy(hbm3[k].at[pl.ds(off, SB)], ev[which][k],
                                  sem_e[which]).wait()

    def _fire_gather(j):
        for b in range(NBUF):
            @pl.when(j % NBUF == b)
            def _(b=b):
                pltpu.async_copy(h_hbm.at[gidx.at[j % RING]], rows[b],
                                 gsem[b])

    def _wait_gather_fire_add(j):
        for b in range(NBUF):
            @pl.when(j % NBUF == b)
            def _(b=b):
                pltpu.make_async_copy(h_hbm.at[gidx.at[j % RING]], rows[b],
                                      gsem[b]).wait()
                pltpu.async_copy(rows[b], acc.at[dslot.at[j % RING]],
                                 asem[b], add=True)

    def _wait_add(j):
        for b in range(NBUF):
            @pl.when(j % NBUF == b)
            def _(b=b):
                pltpu.make_async_copy(rows[b], acc.at[dslot.at[j % RING]],
                                      asem[b]).wait()

    def _pass_body(p, _carry):
        chunk = p * NC + c
        lo = chunk * CN

        with jax.named_scope("sc_zero"):
            pltpu.sync_copy(zeros_hbm, acc.at[pl.ds(s * zrows, zrows)])
            plsc.subcore_barrier()

        ptrv = jnp.zeros((16,), jnp.int32)
        ptr = jnp.int32(0)
        nfired = jnp.int32(0)
        _fire_stage(0, 0)
        for sb in range(NSB):
            which = sb % 2
            if sb + 1 < NSB:
                _fire_stage(sb + 1, 1 - which)
            _wait_stage(sb, which)
            ev_b, ev_e, ev_t = ev[which]

            def _vec(i, ptrv, ev_b=ev_b, ev_e=ev_e, ev_t=ev_t):
                b16 = ev_b[pl.ds(i * 16, 16)]
                e16 = ev_e[pl.ds(i * 16, 16)]
                t16 = ev_t[pl.ds(i * 16, 16)]
                m = (e16 >= lo) & (e16 < lo + CN)
                mi = m.astype(jnp.int32)
                pos = jnp.maximum(ptrv + jnp.cumsum(mi) - 1, 0)
                row = (pos // GB) % RING
                col = pos % GB
                d16 = (e16 - lo) * NBT + (t16 % NBT)
                plsc.store_scatter(gidx, [row, col], b16, mask=m)
                plsc.store_scatter(dslot, [row, col], d16, mask=m)
                return ptrv + plsc.all_reduce_population_count(m)

            with jax.named_scope("sc_filter"):
                ptrv = lax.fori_loop(0, SB // 16, _vec, ptrv)
            ptr = jnp.max(ptrv)

            def _adv(j, _):
                @pl.when(j >= 1)
                def _():
                    _wait_gather_fire_add(j - 1)

                @pl.when(j >= NBUF)
                def _():
                    _wait_add(j - NBUF)
                _fire_gather(j)
                return 0

            with jax.named_scope("sc_advance"):
                lax.fori_loop(nfired, ptr // GB, _adv, 0)
            nfired = ptr // GB

        nb = (ptr + GB - 1) // GB
        total = nb * GB

        def _pad(v, _):
            posv = ptr + v * 16 + lax.iota(jnp.int32, 16)
            m = posv < total
            row = (posv // GB) % RING
            col = posv % GB
            plsc.store_scatter(gidx, [row, col], jnp.zeros((16,), jnp.int32),
                               mask=m)
            jrow = JUNK + lax.rem(posv + s * 8, jnp.int32(ACC_ROWS - ROWS))
            plsc.store_scatter(dslot, [row, col], jrow, mask=m)
            return 0

        with jax.named_scope("sc_pad"):
            lax.fori_loop(0, GB // 16, _pad, 0)

        def _adv2(j, _):
            @pl.when(j >= 1)
            def _():
                _wait_gather_fire_add(j - 1)

            @pl.when(j >= NBUF)
            def _():
                _wait_add(j - NBUF)
            _fire_gather(j)
            return 0

        with jax.named_scope("sc_tail"):
            lax.fori_loop(nfired, nb, _adv2, 0)

            @pl.when(nb > 0)
            def _():
                _wait_gather_fire_add(nb - 1)

            def _drain(j, _):
                _wait_add(j)
                return 0

            lax.fori_loop(jnp.maximum(nb - NBUF, 0), nb, _drain, 0)
        plsc.subcore_barrier()

        with jax.named_scope("sc_flush"):
          pltpu.sync_copy(
            acc.at[pl.ds(s * orows8, orows8)],
            buf_hbm.at[pl.ds(chunk * ROWS + s * orows8, orows8)])

        @pl.when(s == NS - 1)
        def _():
            rem = ROWS - NS * orows8
            pltpu.sync_copy(
                acc.at[pl.ds(NS * orows8, rem)],
                buf_hbm.at[pl.ds(chunk * ROWS + NS * orows8, rem)])

        plsc.subcore_barrier()
        return 0

    lax.fori_loop(0, PASSES, _pass_body, 0)


BK = 1000
NBK = N // BK


def _stats_body(h_ref, b_ref, o_ref):
    i = pl.program_id(0)

    @pl.when(i == 0)
    def _():
        o_ref[...] = jnp.zeros_like(o_ref)

    x = jnp.concatenate([h_ref[...], b_ref[...].astype(jnp.float32)],
                        axis=-1)
    o_ref[0:1, :] += jnp.sum(x, axis=0, keepdims=True)
    o_ref[1:2, :] += jnp.sum(x * x, axis=0, keepdims=True)


def _tc2_stats(h, buf4):
    cd = F + NBT * F
    return pl.pallas_call(
        _stats_body,
        grid=(NBK,),
        in_specs=[
            pl.BlockSpec((BK, F), lambda i: (i, 0)),
            pl.BlockSpec((BK, NBT * F), lambda i: (i, 0)),
        ],
        out_specs=pl.BlockSpec((8, cd), lambda i: (0, 0)),
        out_shape=jax.ShapeDtypeStruct((8, cd), jnp.float32),
    )(h, buf4)


def _apply_body(h_ref, b_ref, st_ref, g_ref, bt_ref, w_ref, o_ref):
    x = jnp.concatenate([h_ref[...], b_ref[...].astype(jnp.float32)],
                        axis=-1)
    m = st_ref[0:1, :] * (1.0 / N)
    v = st_ref[1:2, :] * (1.0 / N) - m * m
    xn = (x - m) * lax.rsqrt(v + EPS) * g_ref[...][None, :] + bt_ref[...][None, :]
    e = jnp.where(xn > 0, xn, (jnp.exp(xn) - 1.0))
    o_ref[...] = jnp.dot(e, w_ref[...], preferred_element_type=jnp.float32)


def _tc2_apply(h, buf4, stats, g2, b2, w2):
    cd = F + NBT * F
    return pl.pallas_call(
        _apply_body,
        grid=(NBK,),
        in_specs=[
            pl.BlockSpec((BK, F), lambda i: (i, 0)),
            pl.BlockSpec((BK, NBT * F), lambda i: (i, 0)),
            pl.BlockSpec((8, cd), lambda i: (0, 0)),
            pl.BlockSpec((cd,), lambda i: (0,)),
            pl.BlockSpec((cd,), lambda i: (0,)),
            pl.BlockSpec((cd, F), lambda i: (0, 0)),
        ],
        out_specs=pl.BlockSpec((BK, F), lambda i: (i, 0)),
        out_shape=jax.ShapeDtypeStruct((N, F), jnp.float32),
    )(h, buf4, stats, g2, b2, w2)


def kernel(atom_features_list, bond_info, bn_gamma1, bn_beta1, W1,
           bn_gamma2, bn_beta2, W2):
    a0 = atom_features_list[0]
    a1 = atom_features_list[1]
    h = _tc1(a0, a1, bn_gamma1, bn_beta1, W1)

    begin = bond_info[:, 0]
    end = bond_info[:, 1]
    bt = bond_info[:, 2]
    zeros128 = jnp.zeros((ACC_ROWS // NS, F), jnp.bfloat16)
    buf = _sc_msgpass(h.astype(jnp.bfloat16), begin, end, bt, zeros128)
    buf4 = buf.reshape(N, NBT * F)

    stats = _tc2_stats(h, buf4)
    return _tc2_apply(h, buf4, stats, bn_gamma2, bn_beta2, W2)
